# baseline (speedup 1.0000x reference)
import jax
import jax.numpy as jnp
from jax.experimental import pallas as pl

_S = 64.0
_M = 0.4

_ROWS_BLK = 16


def _cosface_tile(labels_ref, x_ref, o_ref):
    x = x_ref[...]
    labels = labels_ref[...]
    col = jax.lax.broadcasted_iota(jnp.int32, x.shape, 1)
    mask = (col == labels) & (labels >= 0)
    o_ref[...] = jnp.where(mask, x * _S - (_M * _S), x * _S)


def kernel(logits, labels):
    b, c = logits.shape
    labels2 = labels.astype(jnp.int32).reshape(b, 1)
    grid = (pl.cdiv(b, _ROWS_BLK),)
    return pl.pallas_call(
        _cosface_tile,
        grid=grid,
        in_specs=[
            pl.BlockSpec((_ROWS_BLK, 1), lambda i: (i, 0)),
            pl.BlockSpec((_ROWS_BLK, c), lambda i: (i, 0)),
        ],
        out_specs=pl.BlockSpec((_ROWS_BLK, c), lambda i: (i, 0)),
        out_shape=jax.ShapeDtypeStruct((b, c), logits.dtype),
    )(labels2, logits)


# speedup vs baseline: 3.7836x; 3.7836x over previous
import jax
import jax.numpy as jnp
from jax.experimental import pallas as pl

_S = 64.0
_M = 0.4

_CLS_BLK = 1024


def _cosface_tile(labels_ref, x_ref, o_ref):
    i = pl.program_id(0)
    base = i * _CLS_BLK
    x = x_ref[...]
    labels = labels_ref[...]
    cls = jax.lax.broadcasted_iota(jnp.int32, x.shape, 0) + base
    mask = (cls == labels) & (labels >= 0)
    o_ref[...] = jnp.where(mask, x * _S - (_M * _S), x * _S)


def kernel(logits, labels):
    b, c = logits.shape
    x_t = logits.T
    labels2 = labels.astype(jnp.int32).reshape(1, b)
    grid = (pl.cdiv(c, _CLS_BLK),)
    out_t = pl.pallas_call(
        _cosface_tile,
        grid=grid,
        in_specs=[
            pl.BlockSpec((1, b), lambda i: (0, 0)),
            pl.BlockSpec((_CLS_BLK, b), lambda i: (i, 0)),
        ],
        out_specs=pl.BlockSpec((_CLS_BLK, b), lambda i: (i, 0)),
        out_shape=jax.ShapeDtypeStruct((c, b), logits.dtype),
    )(labels2, x_t)
    return out_t.T


# speedup vs baseline: 3.8490x; 1.0173x over previous
import jax
import jax.numpy as jnp
from jax.experimental import pallas as pl

_S = 64.0
_M = 0.4

_CLS_BLK = 2048


def _cosface_tile(labels_ref, x_ref, o_ref):
    i = pl.program_id(0)
    base = i * _CLS_BLK
    x = x_ref[...]
    labels = labels_ref[...]
    cls = jax.lax.broadcasted_iota(jnp.int32, x.shape, 0) + base
    mask = (cls == labels) & (labels >= 0)
    o_ref[...] = jnp.where(mask, x * _S - (_M * _S), x * _S)


def kernel(logits, labels):
    b, c = logits.shape
    x_t = logits.T
    labels2 = labels.astype(jnp.int32).reshape(1, b)
    grid = (pl.cdiv(c, _CLS_BLK),)
    out_t = pl.pallas_call(
        _cosface_tile,
        grid=grid,
        in_specs=[
            pl.BlockSpec((1, b), lambda i: (0, 0)),
            pl.BlockSpec((_CLS_BLK, b), lambda i: (i, 0)),
        ],
        out_specs=pl.BlockSpec((_CLS_BLK, b), lambda i: (i, 0)),
        out_shape=jax.ShapeDtypeStruct((c, b), logits.dtype),
    )(labels2, x_t)
    return out_t.T


# speedup vs baseline: 3.8496x; 1.0002x over previous
import jax
import jax.numpy as jnp
from jax.experimental import pallas as pl

_S = 64.0
_M = 0.4

_CLS_BLK = 3072


def _cosface_tile(labels_ref, x_ref, o_ref):
    i = pl.program_id(0)
    base = i * _CLS_BLK
    x = x_ref[...]
    labels = labels_ref[...]
    cls = jax.lax.broadcasted_iota(jnp.int32, x.shape, 0) + base
    mask = (cls == labels) & (labels >= 0)
    o_ref[...] = jnp.where(mask, x * _S - (_M * _S), x * _S)


def kernel(logits, labels):
    b, c = logits.shape
    x_t = logits.T
    labels2 = labels.astype(jnp.int32).reshape(1, b)
    grid = (pl.cdiv(c, _CLS_BLK),)
    out_t = pl.pallas_call(
        _cosface_tile,
        grid=grid,
        in_specs=[
            pl.BlockSpec((1, b), lambda i: (0, 0)),
            pl.BlockSpec((_CLS_BLK, b), lambda i: (i, 0)),
        ],
        out_specs=pl.BlockSpec((_CLS_BLK, b), lambda i: (i, 0)),
        out_shape=jax.ShapeDtypeStruct((c, b), logits.dtype),
    )(labels2, x_t)
    return out_t.T
